# Initial kernel scaffold; baseline (speedup 1.0000x reference)
import jax
import jax.numpy as jnp
from jax.experimental import pallas as pl


def kernel(feats, idxs, ds_W, ds_b, W1, b1, gamma, beta, W2, b2):
    raise NotImplementedError("write your pallas kernel here")


# speedup vs baseline: 34.3505x; 34.3505x over previous
import functools

import jax
import jax.numpy as jnp
from jax import lax
from jax.experimental import pallas as pl
from jax.experimental.pallas import tpu as pltpu
from jax.experimental.pallas import tpu_sc as plsc

_B, _C, _N, _K = 2, 256, 16384, 16
_DS, _OUT = 8, 128


_NB1 = 2048


def _ds_kernel(feats_ref, dsw_ref, dsb_ref, f_ref):
    fb = feats_ref[0]
    w = dsw_ref[...]
    r = lax.dot_general(fb, w, (((0,), (1,)), ((), ())),
                        precision=lax.Precision.HIGHEST,
                        preferred_element_type=jnp.float32)
    f_ref[0] = r + dsb_ref[...]


def _make_f(feats, ds_W, ds_b2):
    return pl.pallas_call(
        _ds_kernel,
        grid=(_B, _N // _NB1),
        in_specs=[
            pl.BlockSpec((1, _C, _NB1), lambda b, t: (b, 0, t)),
            pl.BlockSpec((_DS, _C), lambda b, t: (0, 0)),
            pl.BlockSpec((1, _DS), lambda b, t: (0, 0)),
        ],
        out_specs=pl.BlockSpec((1, _NB1, _DS), lambda b, t: (b, t, 0)),
        out_shape=jax.ShapeDtypeStruct((_B, _N, _DS), jnp.float32),
    )(feats, ds_W, ds_b2)


_NW = 32
_CH = 128
_CPW = (_B * _N * _K) // (_NW * _CH)


def _sc_gather(table, idx3):
    mesh = plsc.VectorSubcoreMesh(core_axis_name="c", subcore_axis_name="s")

    @functools.partial(
        pl.kernel,
        mesh=mesh,
        out_type=jax.ShapeDtypeStruct((_B * _N * _K, _DS), jnp.float32),
        compiler_params=pltpu.CompilerParams(use_tc_tiling_on_sc=False),
        scratch_types=[
            pltpu.VMEM((_CPW, _CH), jnp.int32),
            pltpu.VMEM((2, _CH, _DS), jnp.float32),
            pltpu.SemaphoreType.DMA,
            pltpu.SemaphoreType.DMA,
        ],
    )
    def k(table_hbm, idx_hbm, nn_hbm, idx_v, rows_v, sem0, sem1):
        wid = lax.axis_index("s") * 2 + lax.axis_index("c")
        pltpu.sync_copy(idx_hbm.at[wid], idx_v)
        base = wid * (_CPW * _CH)
        sems = (sem0, sem1)

        def start(j, slot):
            pltpu.async_copy(table_hbm.at[idx_v.at[j]], rows_v.at[slot],
                             sems[slot])

        def finish(j, slot):
            pltpu.make_async_copy(table_hbm.at[idx_v.at[j]], rows_v.at[slot],
                                  sems[slot]).wait()
            pltpu.sync_copy(rows_v.at[slot],
                            nn_hbm.at[pl.ds(base + j * _CH, _CH)])

        start(0, 0)

        def body(p, _):
            j = 2 * p
            start(j + 1, 1)
            finish(j, 0)
            start(j + 2, 0)
            finish(j + 1, 1)
            return 0

        lax.fori_loop(0, _CPW // 2 - 1, body, 0)
        start(_CPW - 1, 1)
        finish(_CPW - 2, 0)
        finish(_CPW - 1, 1)

    return k(table, idx3)


_NB = 256


def _stats_kernel(f_ref, nn_ref, w1_ref, b1_ref, o_ref):
    step = pl.program_id(0) * pl.num_programs(1) + pl.program_id(1)

    @pl.when(step == 0)
    def _():
        o_ref[...] = jnp.zeros_like(o_ref)

    f = f_ref[0]
    nn = nn_ref[0]
    w1 = w1_ref[...]
    gc = lax.dot_general(f, w1, (((1,), (1,)), ((), ())),
                         precision=lax.Precision.HIGHEST,
                         preferred_element_type=jnp.float32)
    gn = lax.dot_general(nn, w1, (((1,), (1,)), ((), ())),
                         precision=lax.Precision.HIGHEST,
                         preferred_element_type=jnp.float32)
    h = (gc[:, None, :] - gn.reshape(_NB, _K, _OUT)) + b1_ref[...][None, :, :]
    s1 = jnp.sum(h, axis=(0, 1))
    s2 = jnp.sum(h * h, axis=(0, 1))
    o_ref[...] += jnp.stack([s1, s2])


def _stats(f, nn3, W1, b1_2):
    return pl.pallas_call(
        _stats_kernel,
        grid=(_B, _N // _NB),
        in_specs=[
            pl.BlockSpec((1, _NB, _DS), lambda b, t: (b, t, 0)),
            pl.BlockSpec((1, _NB * _K, _DS), lambda b, t: (b, t, 0)),
            pl.BlockSpec((_OUT, _DS), lambda b, t: (0, 0)),
            pl.BlockSpec((1, _OUT), lambda b, t: (0, 0)),
        ],
        out_specs=pl.BlockSpec((2, _OUT), lambda b, t: (0, 0)),
        out_shape=jax.ShapeDtypeStruct((2, _OUT), jnp.float32),
    )(f, nn3, W1, b1_2)


def _main_kernel(f_ref, nn_ref, w1_ref, b1_ref, ae_ref, w2_ref, b2_ref,
                 o_ref):
    f = f_ref[0]
    nn = nn_ref[0]
    w1 = w1_ref[...]
    gc = lax.dot_general(f, w1, (((1,), (1,)), ((), ())),
                         precision=lax.Precision.HIGHEST,
                         preferred_element_type=jnp.float32)
    gn = lax.dot_general(nn, w1, (((1,), (1,)), ((), ())),
                         precision=lax.Precision.HIGHEST,
                         preferred_element_type=jnp.float32)
    h = (gc[:, None, :] - gn.reshape(_NB, _K, _OUT)) + b1_ref[...][None, :, :]
    a = ae_ref[0][None, None, :]
    e = ae_ref[1][None, None, :]
    r = jnp.maximum(h * a + e, 0.0)
    y = lax.dot_general(r.reshape(_NB * _K, _OUT), w2_ref[...],
                        (((1,), (1,)), ((), ())),
                        precision=lax.Precision.HIGHEST,
                        preferred_element_type=jnp.float32)
    ym = jnp.max(y.reshape(_NB, _K, _OUT), axis=1) + b2_ref[...]
    o_ref[0] = ym.T


def _main(f, nn3, W1, b1_2, ae, W2, b2_2):
    return pl.pallas_call(
        _main_kernel,
        grid=(_B, _N // _NB),
        in_specs=[
            pl.BlockSpec((1, _NB, _DS), lambda b, t: (b, t, 0)),
            pl.BlockSpec((1, _NB * _K, _DS), lambda b, t: (b, t, 0)),
            pl.BlockSpec((_OUT, _DS), lambda b, t: (0, 0)),
            pl.BlockSpec((1, _OUT), lambda b, t: (0, 0)),
            pl.BlockSpec((2, _OUT), lambda b, t: (0, 0)),
            pl.BlockSpec((_OUT, _OUT), lambda b, t: (0, 0)),
            pl.BlockSpec((1, _OUT), lambda b, t: (0, 0)),
        ],
        out_specs=pl.BlockSpec((1, _OUT, _NB), lambda b, t: (b, 0, t)),
        out_shape=jax.ShapeDtypeStruct((_B, _OUT, _N), jnp.float32),
    )(f, nn3, W1, b1_2, ae, W2, b2_2)


def kernel(feats, idxs, ds_W, ds_b, W1, b1, gamma, beta, W2, b2):
    feats3 = feats.reshape(_B, _C, _N)
    f = _make_f(feats3, ds_W, ds_b.reshape(1, _DS))

    idx32 = (idxs.astype(jnp.int32)
             + (jnp.arange(_B, dtype=jnp.int32) * _N)[:, None, None])
    idx3 = idx32.reshape(_NW, _CPW, _CH)
    nn = _sc_gather(f.reshape(_B * _N, _DS), idx3)
    nn3 = nn.reshape(_B, _N * _K, _DS)

    b1_2 = b1.reshape(1, _OUT)
    st = _stats(f, nn3, W1, b1_2)
    cnt = float(_B * _N * _K)
    mean = st[0] / cnt
    var = st[1] / cnt - mean * mean
    a = gamma * lax.rsqrt(var + 1e-5)
    e = beta - mean * a
    ae = jnp.stack([a, e])

    out = _main(f, nn3, W1, b1_2, ae, W2, b2.reshape(1, _OUT))
    return out.reshape(_B, _OUT, _N, 1)


# speedup vs baseline: 54.4537x; 1.5852x over previous
import functools

import jax
import jax.numpy as jnp
from jax import lax
from jax.experimental import pallas as pl
from jax.experimental.pallas import tpu as pltpu
from jax.experimental.pallas import tpu_sc as plsc

_B, _C, _N, _K = 2, 256, 16384, 16
_DS, _OUT = 8, 128
_HP = lax.Precision.HIGHEST


_NB1 = 2048


def _ds_kernel(feats_ref, dsw_ref, dsb_ref, w1_ref, f_ref, gc_ref):
    fb = feats_ref[0]
    w = dsw_ref[...]
    r = lax.dot_general(fb, w, (((0,), (1,)), ((), ())),
                        precision=_HP,
                        preferred_element_type=jnp.float32)
    r = r + dsb_ref[...]
    f_ref[0] = r
    gc_ref[0] = lax.dot_general(r, w1_ref[...], (((1,), (1,)), ((), ())),
                                precision=_HP,
                                preferred_element_type=jnp.float32)


def _make_f(feats, ds_W, ds_b2, W1):
    return pl.pallas_call(
        _ds_kernel,
        grid=(_B, _N // _NB1),
        in_specs=[
            pl.BlockSpec((1, _C, _NB1), lambda b, t: (b, 0, t)),
            pl.BlockSpec((_DS, _C), lambda b, t: (0, 0)),
            pl.BlockSpec((1, _DS), lambda b, t: (0, 0)),
            pl.BlockSpec((_OUT, _DS), lambda b, t: (0, 0)),
        ],
        out_specs=[
            pl.BlockSpec((1, _NB1, _DS), lambda b, t: (b, t, 0)),
            pl.BlockSpec((1, _NB1, _OUT), lambda b, t: (b, t, 0)),
        ],
        out_shape=[
            jax.ShapeDtypeStruct((_B, _N, _DS), jnp.float32),
            jax.ShapeDtypeStruct((_B, _N, _OUT), jnp.float32),
        ],
    )(feats, ds_W, ds_b2, W1)


_NW = 32
_CH = 128
_CPW = (_B * _N * _K) // (_NW * _CH)


def _sc_gather(table, idx3):
    mesh = plsc.VectorSubcoreMesh(core_axis_name="c", subcore_axis_name="s")

    @functools.partial(
        pl.kernel,
        mesh=mesh,
        out_type=jax.ShapeDtypeStruct((_B * _N * _K, _DS), jnp.float32),
        compiler_params=pltpu.CompilerParams(use_tc_tiling_on_sc=False),
        scratch_types=[
            pltpu.VMEM((_CPW, _CH), jnp.int32),
            pltpu.VMEM((2, _CH, _DS), jnp.float32),
            pltpu.SemaphoreType.DMA,
            pltpu.SemaphoreType.DMA,
        ],
    )
    def k(table_hbm, idx_hbm, nn_hbm, idx_v, rows_v, sem0, sem1):
        wid = lax.axis_index("s") * 2 + lax.axis_index("c")
        pltpu.sync_copy(idx_hbm.at[wid], idx_v)
        base = wid * (_CPW * _CH)
        sems = (sem0, sem1)

        def start(j, slot):
            pltpu.async_copy(table_hbm.at[idx_v.at[j]], rows_v.at[slot],
                             sems[slot])

        def finish(j, slot):
            pltpu.make_async_copy(table_hbm.at[idx_v.at[j]], rows_v.at[slot],
                                  sems[slot]).wait()
            pltpu.sync_copy(rows_v.at[slot],
                            nn_hbm.at[pl.ds(base + j * _CH, _CH)])

        start(0, 0)

        def body(p, _):
            j = 2 * p
            start(j + 1, 1)
            finish(j, 0)
            start(j + 2, 0)
            finish(j + 1, 1)
            return 0

        lax.fori_loop(0, _CPW // 2 - 1, body, 0)
        start(_CPW - 1, 1)
        finish(_CPW - 2, 0)
        finish(_CPW - 1, 1)

    return k(table, idx3)


_NB = 256


def _stats_kernel(gc_ref, nn_ref, bd_ref, b1_ref, o_ref):
    step = pl.program_id(0) * pl.num_programs(1) + pl.program_id(1)

    @pl.when(step == 0)
    def _():
        o_ref[...] = jnp.zeros_like(o_ref)

    gc = gc_ref[0]
    nn = nn_ref[0]
    gn = lax.dot_general(nn, bd_ref[...], (((1,), (0,)), ((), ())),
                         precision=_HP,
                         preferred_element_type=jnp.float32)
    gcb = gc + b1_ref[...]
    s1 = jnp.zeros((_OUT,), jnp.float32)
    s2 = jnp.zeros((_OUT,), jnp.float32)
    for m in range(_K):
        h = gcb - gn[:, m * _OUT:(m + 1) * _OUT]
        s1 = s1 + jnp.sum(h, axis=0)
        s2 = s2 + jnp.sum(h * h, axis=0)
    o_ref[...] += jnp.stack([s1, s2])


def _stats(gc, nn128, bd, b1_2):
    return pl.pallas_call(
        _stats_kernel,
        grid=(_B, _N // _NB),
        in_specs=[
            pl.BlockSpec((1, _NB, _OUT), lambda b, t: (b, t, 0)),
            pl.BlockSpec((1, _NB, _K * _DS), lambda b, t: (b, t, 0)),
            pl.BlockSpec((_K * _DS, _K * _OUT), lambda b, t: (0, 0)),
            pl.BlockSpec((1, _OUT), lambda b, t: (0, 0)),
        ],
        out_specs=pl.BlockSpec((2, _OUT), lambda b, t: (0, 0)),
        out_shape=jax.ShapeDtypeStruct((2, _OUT), jnp.float32),
    )(gc, nn128, bd, b1_2)


def _main_kernel(gc_ref, nn_ref, bd_ref, b1_ref, ae_ref, w2_ref, b2_ref,
                 o_ref):
    gc = gc_ref[0]
    nn = nn_ref[0]
    gn = lax.dot_general(nn, bd_ref[...], (((1,), (0,)), ((), ())),
                         precision=_HP,
                         preferred_element_type=jnp.float32)
    a = ae_ref[0][None, :]
    e = ae_ref[1][None, :]
    gcb = (gc + b1_ref[...]) * a + e
    rs = []
    for m in range(_K):
        h = gcb - gn[:, m * _OUT:(m + 1) * _OUT] * a
        rs.append(jnp.maximum(h, 0.0))
    r = jnp.concatenate(rs, axis=0)
    y = lax.dot_general(r, w2_ref[...], (((1,), (1,)), ((), ())),
                        precision=_HP,
                        preferred_element_type=jnp.float32)
    ym = jnp.max(y.reshape(_K, _NB, _OUT), axis=0) + b2_ref[...]
    o_ref[0] = ym.T


def _main(gc, nn128, bd, b1_2, ae, W2, b2_2):
    return pl.pallas_call(
        _main_kernel,
        grid=(_B, _N // _NB),
        in_specs=[
            pl.BlockSpec((1, _NB, _OUT), lambda b, t: (b, t, 0)),
            pl.BlockSpec((1, _NB, _K * _DS), lambda b, t: (b, t, 0)),
            pl.BlockSpec((_K * _DS, _K * _OUT), lambda b, t: (0, 0)),
            pl.BlockSpec((1, _OUT), lambda b, t: (0, 0)),
            pl.BlockSpec((2, _OUT), lambda b, t: (0, 0)),
            pl.BlockSpec((_OUT, _OUT), lambda b, t: (0, 0)),
            pl.BlockSpec((1, _OUT), lambda b, t: (0, 0)),
        ],
        out_specs=pl.BlockSpec((1, _OUT, _NB), lambda b, t: (b, 0, t)),
        out_shape=jax.ShapeDtypeStruct((_B, _OUT, _N), jnp.float32),
    )(gc, nn128, bd, b1_2, ae, W2, b2_2)


def kernel(feats, idxs, ds_W, ds_b, W1, b1, gamma, beta, W2, b2):
    feats3 = feats.reshape(_B, _C, _N)
    f, gc = _make_f(feats3, ds_W, ds_b.reshape(1, _DS), W1)

    idx32 = (idxs.astype(jnp.int32)
             + (jnp.arange(_B, dtype=jnp.int32) * _N)[:, None, None])
    idx3 = idx32.reshape(_NW, _CPW, _CH)
    nn = _sc_gather(f.reshape(_B * _N, _DS), idx3)
    nn128 = nn.reshape(_B, _N, _K * _DS)

    bd = (jnp.eye(_K, dtype=jnp.float32)[:, None, :, None]
          * W1.T[None, :, None, :]).reshape(_K * _DS, _K * _OUT)

    b1_2 = b1.reshape(1, _OUT)
    st = _stats(gc, nn128, bd, b1_2)
    cnt = float(_B * _N * _K)
    mean = st[0] / cnt
    var = st[1] / cnt - mean * mean
    a = gamma * lax.rsqrt(var + 1e-5)
    e = beta - mean * a
    ae = jnp.stack([a, e])

    out = _main(gc, nn128, bd, b1_2, ae, W2, b2.reshape(1, _OUT))
    return out.reshape(_B, _OUT, _N, 1)


# speedup vs baseline: 125.6273x; 2.3070x over previous
import functools

import jax
import jax.numpy as jnp
from jax import lax
from jax.experimental import pallas as pl
from jax.experimental.pallas import tpu as pltpu
from jax.experimental.pallas import tpu_sc as plsc

_B, _C, _N, _K = 2, 256, 16384, 16
_DS, _OUT = 8, 128
_HP = lax.Precision.DEFAULT


_NB1 = 2048


def _ds_kernel(feats_ref, dsw_ref, dsb_ref, w1_ref, f_ref, gc_ref):
    fb = feats_ref[0]
    w = dsw_ref[...]
    r = lax.dot_general(fb, w, (((0,), (1,)), ((), ())),
                        precision=_HP,
                        preferred_element_type=jnp.float32)
    r = r + dsb_ref[...]
    f_ref[0] = r
    gc_ref[0] = lax.dot_general(r, w1_ref[...], (((1,), (1,)), ((), ())),
                                precision=_HP,
                                preferred_element_type=jnp.float32)


def _make_f(feats, ds_W, ds_b2, W1):
    return pl.pallas_call(
        _ds_kernel,
        grid=(_B, _N // _NB1),
        in_specs=[
            pl.BlockSpec((1, _C, _NB1), lambda b, t: (b, 0, t)),
            pl.BlockSpec((_DS, _C), lambda b, t: (0, 0)),
            pl.BlockSpec((1, _DS), lambda b, t: (0, 0)),
            pl.BlockSpec((_OUT, _DS), lambda b, t: (0, 0)),
        ],
        out_specs=[
            pl.BlockSpec((1, _NB1, _DS), lambda b, t: (b, t, 0)),
            pl.BlockSpec((1, _NB1, _OUT), lambda b, t: (b, t, 0)),
        ],
        out_shape=[
            jax.ShapeDtypeStruct((_B, _N, _DS), jnp.float32),
            jax.ShapeDtypeStruct((_B, _N, _OUT), jnp.float32),
        ],
    )(feats, ds_W, ds_b2, W1)


_NW = 32
_CH = 128
_CPW = (_B * _N * _K) // (_NW * _CH)


def _sc_gather(table, idx3):
    mesh = plsc.VectorSubcoreMesh(core_axis_name="c", subcore_axis_name="s")

    @functools.partial(
        pl.kernel,
        mesh=mesh,
        out_type=jax.ShapeDtypeStruct((_B * _N * _K, _DS), jnp.float32),
        compiler_params=pltpu.CompilerParams(use_tc_tiling_on_sc=False),
        scratch_types=[
            pltpu.VMEM((_CPW, _CH), jnp.int32),
            pltpu.VMEM((2, _CH, _DS), jnp.float32),
            pltpu.SemaphoreType.DMA,
            pltpu.SemaphoreType.DMA,
        ],
    )
    def k(table_hbm, idx_hbm, nn_hbm, idx_v, rows_v, sem0, sem1):
        wid = lax.axis_index("s") * 2 + lax.axis_index("c")
        pltpu.sync_copy(idx_hbm.at[wid], idx_v)
        base = wid * (_CPW * _CH)
        sems = (sem0, sem1)

        def start(j, slot):
            pltpu.async_copy(table_hbm.at[idx_v.at[j]], rows_v.at[slot],
                             sems[slot])

        def finish(j, slot):
            pltpu.make_async_copy(table_hbm.at[idx_v.at[j]], rows_v.at[slot],
                                  sems[slot]).wait()
            pltpu.sync_copy(rows_v.at[slot],
                            nn_hbm.at[pl.ds(base + j * _CH, _CH)])

        start(0, 0)

        def body(p, _):
            j = 2 * p
            start(j + 1, 1)
            finish(j, 0)
            start(j + 2, 0)
            finish(j + 1, 1)
            return 0

        lax.fori_loop(0, _CPW // 2 - 1, body, 0)
        start(_CPW - 1, 1)
        finish(_CPW - 2, 0)
        finish(_CPW - 1, 1)

    return k(table, idx3)


_NB = 256


def _stats_kernel(gc_ref, nn_ref, bd_ref, b1_ref, o_ref):
    step = pl.program_id(0) * pl.num_programs(1) + pl.program_id(1)

    @pl.when(step == 0)
    def _():
        o_ref[...] = jnp.zeros_like(o_ref)

    gc = gc_ref[0]
    nn = nn_ref[0]
    gn = lax.dot_general(nn, bd_ref[...], (((1,), (0,)), ((), ())),
                         precision=_HP,
                         preferred_element_type=jnp.float32)
    gcb = gc + b1_ref[...]
    s1 = jnp.zeros((_OUT,), jnp.float32)
    s2 = jnp.zeros((_OUT,), jnp.float32)
    for m in range(_K):
        h = gcb - gn[:, m * _OUT:(m + 1) * _OUT]
        s1 = s1 + jnp.sum(h, axis=0)
        s2 = s2 + jnp.sum(h * h, axis=0)
    o_ref[...] += jnp.stack([s1, s2])


def _stats(gc, nn128, bd, b1_2):
    return pl.pallas_call(
        _stats_kernel,
        grid=(_B, _N // _NB),
        in_specs=[
            pl.BlockSpec((1, _NB, _OUT), lambda b, t: (b, t, 0)),
            pl.BlockSpec((1, _NB, _K * _DS), lambda b, t: (b, t, 0)),
            pl.BlockSpec((_K * _DS, _K * _OUT), lambda b, t: (0, 0)),
            pl.BlockSpec((1, _OUT), lambda b, t: (0, 0)),
        ],
        out_specs=pl.BlockSpec((2, _OUT), lambda b, t: (0, 0)),
        out_shape=jax.ShapeDtypeStruct((2, _OUT), jnp.float32),
    )(gc, nn128, bd, b1_2)


def _main_kernel(gc_ref, nn_ref, bd_ref, b1_ref, ae_ref, w2_ref, b2_ref,
                 o_ref):
    gc = gc_ref[0]
    nn = nn_ref[0]
    gn = lax.dot_general(nn, bd_ref[...], (((1,), (0,)), ((), ())),
                         precision=_HP,
                         preferred_element_type=jnp.float32)
    a = ae_ref[0][None, :]
    e = ae_ref[1][None, :]
    gcb = (gc + b1_ref[...]) * a + e
    rs = []
    for m in range(_K):
        h = gcb - gn[:, m * _OUT:(m + 1) * _OUT] * a
        rs.append(jnp.maximum(h, 0.0))
    r = jnp.concatenate(rs, axis=0)
    y = lax.dot_general(r, w2_ref[...], (((1,), (1,)), ((), ())),
                        precision=_HP,
                        preferred_element_type=jnp.float32)
    ym = jnp.max(y.reshape(_K, _NB, _OUT), axis=0) + b2_ref[...]
    o_ref[0] = ym.T


def _main(gc, nn128, bd, b1_2, ae, W2, b2_2):
    return pl.pallas_call(
        _main_kernel,
        grid=(_B, _N // _NB),
        in_specs=[
            pl.BlockSpec((1, _NB, _OUT), lambda b, t: (b, t, 0)),
            pl.BlockSpec((1, _NB, _K * _DS), lambda b, t: (b, t, 0)),
            pl.BlockSpec((_K * _DS, _K * _OUT), lambda b, t: (0, 0)),
            pl.BlockSpec((1, _OUT), lambda b, t: (0, 0)),
            pl.BlockSpec((2, _OUT), lambda b, t: (0, 0)),
            pl.BlockSpec((_OUT, _OUT), lambda b, t: (0, 0)),
            pl.BlockSpec((1, _OUT), lambda b, t: (0, 0)),
        ],
        out_specs=pl.BlockSpec((1, _OUT, _NB), lambda b, t: (b, 0, t)),
        out_shape=jax.ShapeDtypeStruct((_B, _OUT, _N), jnp.float32),
    )(gc, nn128, bd, b1_2, ae, W2, b2_2)


def kernel(feats, idxs, ds_W, ds_b, W1, b1, gamma, beta, W2, b2):
    feats3 = feats.reshape(_B, _C, _N)
    f, gc = _make_f(feats3, ds_W, ds_b.reshape(1, _DS), W1)

    idx32 = (idxs.astype(jnp.int32)
             + (jnp.arange(_B, dtype=jnp.int32) * _N)[:, None, None])
    idx3 = idx32.reshape(_NW, _CPW, _CH)
    nn = _sc_gather(f.reshape(_B * _N, _DS), idx3)
    nn128 = nn.reshape(_B, _N, _K * _DS)

    bd = (jnp.eye(_K, dtype=jnp.float32)[:, None, :, None]
          * W1.T[None, :, None, :]).reshape(_K * _DS, _K * _OUT)

    b1_2 = b1.reshape(1, _OUT)
    st = _stats(gc, nn128, bd, b1_2)
    cnt = float(_B * _N * _K)
    mean = st[0] / cnt
    var = st[1] / cnt - mean * mean
    a = gamma * lax.rsqrt(var + 1e-5)
    e = beta - mean * a
    ae = jnp.stack([a, e])

    out = _main(gc, nn128, bd, b1_2, ae, W2, b2.reshape(1, _OUT))
    return out.reshape(_B, _OUT, _N, 1)


# speedup vs baseline: 125.8405x; 1.0017x over previous
import functools

import jax
import jax.numpy as jnp
from jax import lax
from jax.experimental import pallas as pl
from jax.experimental.pallas import tpu as pltpu
from jax.experimental.pallas import tpu_sc as plsc

_B, _C, _N, _K = 2, 256, 16384, 16
_DS, _OUT = 8, 128
_HP = lax.Precision.DEFAULT


_NB1 = 2048


def _ds_kernel(feats_ref, dsw_ref, dsb_ref, w1_ref, f_ref, gc_ref):
    fb = feats_ref[0]
    w = dsw_ref[...]
    r = lax.dot_general(fb, w, (((0,), (1,)), ((), ())),
                        precision=_HP,
                        preferred_element_type=jnp.float32)
    r = r + dsb_ref[...]
    f_ref[0] = r
    gc_ref[0] = lax.dot_general(r, w1_ref[...], (((1,), (1,)), ((), ())),
                                precision=_HP,
                                preferred_element_type=jnp.float32)


def _make_f(feats, ds_W, ds_b2, W1):
    return pl.pallas_call(
        _ds_kernel,
        grid=(_B, _N // _NB1),
        in_specs=[
            pl.BlockSpec((1, _C, _NB1), lambda b, t: (b, 0, t)),
            pl.BlockSpec((_DS, _C), lambda b, t: (0, 0)),
            pl.BlockSpec((1, _DS), lambda b, t: (0, 0)),
            pl.BlockSpec((_OUT, _DS), lambda b, t: (0, 0)),
        ],
        out_specs=[
            pl.BlockSpec((1, _NB1, _DS), lambda b, t: (b, t, 0)),
            pl.BlockSpec((1, _NB1, _OUT), lambda b, t: (b, t, 0)),
        ],
        out_shape=[
            jax.ShapeDtypeStruct((_B, _N, _DS), jnp.float32),
            jax.ShapeDtypeStruct((_B, _N, _OUT), jnp.float32),
        ],
    )(feats, ds_W, ds_b2, W1)


_NW = 32
_CH = 128
_CPW = (_B * _N * _K) // (_NW * _CH)


def _sc_gather(table, idx3):
    mesh = plsc.VectorSubcoreMesh(core_axis_name="c", subcore_axis_name="s")

    @functools.partial(
        pl.kernel,
        mesh=mesh,
        out_type=jax.ShapeDtypeStruct((_B * _N * _K, _DS), jnp.float32),
        compiler_params=pltpu.CompilerParams(use_tc_tiling_on_sc=False),
        scratch_types=[
            pltpu.VMEM((_CPW, _CH), jnp.int32),
            pltpu.VMEM((2, _CH, _DS), jnp.float32),
            pltpu.SemaphoreType.DMA,
            pltpu.SemaphoreType.DMA,
        ],
    )
    def k(table_hbm, idx_hbm, nn_hbm, idx_v, rows_v, sem0, sem1):
        wid = lax.axis_index("s") * 2 + lax.axis_index("c")
        pltpu.sync_copy(idx_hbm.at[wid], idx_v)
        base = wid * (_CPW * _CH)
        sems = (sem0, sem1)

        def start(j, slot):
            pltpu.async_copy(table_hbm.at[idx_v.at[j]], rows_v.at[slot],
                             sems[slot])

        def finish(j, slot):
            pltpu.make_async_copy(table_hbm.at[idx_v.at[j]], rows_v.at[slot],
                                  sems[slot]).wait()
            pltpu.sync_copy(rows_v.at[slot],
                            nn_hbm.at[pl.ds(base + j * _CH, _CH)])

        start(0, 0)

        def body(p, _):
            j = 2 * p
            start(j + 1, 1)
            finish(j, 0)
            start(j + 2, 0)
            finish(j + 1, 1)
            return 0

        lax.fori_loop(0, _CPW // 2 - 1, body, 0)
        start(_CPW - 1, 1)
        finish(_CPW - 2, 0)
        finish(_CPW - 1, 1)

    return k(table, idx3)


_NB = 256


def _stats_kernel(gc_ref, nn_ref, bd_ref, b1_ref, o_ref):
    step = pl.program_id(0) * pl.num_programs(1) + pl.program_id(1)

    @pl.when(step == 0)
    def _():
        o_ref[...] = jnp.zeros_like(o_ref)

    gc = gc_ref[0]
    nn = nn_ref[0].astype(jnp.bfloat16)
    gn = lax.dot_general(nn, bd_ref[...], (((1,), (0,)), ((), ())),
                         precision=_HP,
                         preferred_element_type=jnp.float32)
    gcb = gc + b1_ref[...]
    s1 = jnp.zeros((_OUT,), jnp.float32)
    s2 = jnp.zeros((_OUT,), jnp.float32)
    for m in range(_K):
        h = gcb - gn[:, m * _OUT:(m + 1) * _OUT]
        s1 = s1 + jnp.sum(h, axis=0)
        s2 = s2 + jnp.sum(h * h, axis=0)
    o_ref[...] += jnp.stack([s1, s2])


def _stats(gc, nn128, bd, b1_2):
    return pl.pallas_call(
        _stats_kernel,
        grid=(_B, _N // _NB),
        in_specs=[
            pl.BlockSpec((1, _NB, _OUT), lambda b, t: (b, t, 0)),
            pl.BlockSpec((1, _NB, _K * _DS), lambda b, t: (b, t, 0)),
            pl.BlockSpec((_K * _DS, _K * _OUT), lambda b, t: (0, 0)),
            pl.BlockSpec((1, _OUT), lambda b, t: (0, 0)),
        ],
        out_specs=pl.BlockSpec((2, _OUT), lambda b, t: (0, 0)),
        out_shape=jax.ShapeDtypeStruct((2, _OUT), jnp.float32),
    )(gc, nn128, bd, b1_2)


def _main_kernel(gc_ref, nn_ref, bd_ref, b1_ref, ae_ref, w2p_ref, b2_ref,
                 o_ref):
    gc = gc_ref[0]
    nn = nn_ref[0].astype(jnp.bfloat16)
    gn = lax.dot_general(nn, bd_ref[...], (((1,), (0,)), ((), ())),
                         precision=_HP,
                         preferred_element_type=jnp.float32)
    a = ae_ref[0][None, :]
    e = ae_ref[1][None, :]
    gcb = (gc + b1_ref[...]) * a + e
    rs = []
    for m in range(0, _K, 2):
        h0 = gcb - gn[:, m * _OUT:(m + 1) * _OUT] * a
        h1 = gcb - gn[:, (m + 1) * _OUT:(m + 2) * _OUT] * a
        rs.append(jnp.concatenate(
            [jnp.maximum(h0, 0.0), jnp.maximum(h1, 0.0)],
            axis=1).astype(jnp.bfloat16))
    r = jnp.concatenate(rs, axis=0)
    y = lax.dot_general(r, w2p_ref[...], (((1,), (0,)), ((), ())),
                        precision=_HP,
                        preferred_element_type=jnp.float32)
    ym2 = jnp.max(y.reshape(_K // 2, _NB, 2 * _OUT), axis=0)
    ym = (jnp.maximum(ym2[:, :_OUT], ym2[:, _OUT:]) + b2_ref[...])
    o_ref[0] = ym.T


def _main(gc, nn128, bd, b1_2, ae, w2p, b2_2):
    return pl.pallas_call(
        _main_kernel,
        grid=(_B, _N // _NB),
        in_specs=[
            pl.BlockSpec((1, _NB, _OUT), lambda b, t: (b, t, 0)),
            pl.BlockSpec((1, _NB, _K * _DS), lambda b, t: (b, t, 0)),
            pl.BlockSpec((_K * _DS, _K * _OUT), lambda b, t: (0, 0)),
            pl.BlockSpec((1, _OUT), lambda b, t: (0, 0)),
            pl.BlockSpec((2, _OUT), lambda b, t: (0, 0)),
            pl.BlockSpec((2 * _OUT, 2 * _OUT), lambda b, t: (0, 0)),
            pl.BlockSpec((1, _OUT), lambda b, t: (0, 0)),
        ],
        out_specs=pl.BlockSpec((1, _OUT, _NB), lambda b, t: (b, 0, t)),
        out_shape=jax.ShapeDtypeStruct((_B, _OUT, _N), jnp.float32),
    )(gc, nn128, bd, b1_2, ae, w2p, b2_2)


def kernel(feats, idxs, ds_W, ds_b, W1, b1, gamma, beta, W2, b2):
    feats3 = feats.reshape(_B, _C, _N)
    f, gc = _make_f(feats3, ds_W, ds_b.reshape(1, _DS), W1)

    idx32 = (idxs.astype(jnp.int32)
             + (jnp.arange(_B, dtype=jnp.int32) * _N)[:, None, None])
    idx3 = idx32.reshape(_NW, _CPW, _CH)
    nn = _sc_gather(f.reshape(_B * _N, _DS), idx3)
    nn128 = nn.reshape(_B, _N, _K * _DS)

    bd = (jnp.eye(_K, dtype=jnp.float32)[:, None, :, None]
          * W1.T[None, :, None, :]).reshape(_K * _DS, _K * _OUT)
    bd = bd.astype(jnp.bfloat16)
    z = jnp.zeros((_OUT, _OUT), jnp.float32)
    w2p = jnp.block([[W2.T, z], [z, W2.T]]).astype(jnp.bfloat16)

    b1_2 = b1.reshape(1, _OUT)
    st = _stats(gc, nn128, bd, b1_2)
    cnt = float(_B * _N * _K)
    mean = st[0] / cnt
    var = st[1] / cnt - mean * mean
    a = gamma * lax.rsqrt(var + 1e-5)
    e = beta - mean * a
    ae = jnp.stack([a, e])

    out = _main(gc, nn128, bd, b1_2, ae, w2p, b2.reshape(1, _OUT))
    return out.reshape(_B, _OUT, _N, 1)


# speedup vs baseline: 126.7798x; 1.0075x over previous
import functools

import jax
import jax.numpy as jnp
from jax import lax
from jax.experimental import pallas as pl
from jax.experimental.pallas import tpu as pltpu
from jax.experimental.pallas import tpu_sc as plsc

_B, _C, _N, _K = 2, 256, 16384, 16
_DS, _OUT = 8, 128
_HP = lax.Precision.DEFAULT


_NB1 = 2048


def _ds_kernel(feats_ref, dsw_ref, dsb_ref, w1_ref, f_ref, gc_ref):
    fb = feats_ref[0]
    w = dsw_ref[...]
    r = lax.dot_general(fb, w, (((0,), (1,)), ((), ())),
                        precision=_HP,
                        preferred_element_type=jnp.float32)
    r = r + dsb_ref[...]
    f_ref[0] = r
    gc_ref[0] = lax.dot_general(r, w1_ref[...], (((1,), (1,)), ((), ())),
                                precision=_HP,
                                preferred_element_type=jnp.float32)


def _make_f(feats, ds_W, ds_b2, W1):
    return pl.pallas_call(
        _ds_kernel,
        grid=(_B, _N // _NB1),
        in_specs=[
            pl.BlockSpec((1, _C, _NB1), lambda b, t: (b, 0, t)),
            pl.BlockSpec((_DS, _C), lambda b, t: (0, 0)),
            pl.BlockSpec((1, _DS), lambda b, t: (0, 0)),
            pl.BlockSpec((_OUT, _DS), lambda b, t: (0, 0)),
        ],
        out_specs=[
            pl.BlockSpec((1, _NB1, _DS), lambda b, t: (b, t, 0)),
            pl.BlockSpec((1, _NB1, _OUT), lambda b, t: (b, t, 0)),
        ],
        out_shape=[
            jax.ShapeDtypeStruct((_B, _N, _DS), jnp.float32),
            jax.ShapeDtypeStruct((_B, _N, _OUT), jnp.float32),
        ],
    )(feats, ds_W, ds_b2, W1)


_NW = 32
_CH = 128
_CPW = (_B * _N * _K) // (_NW * _CH)


def _sc_gather(table, idx3):
    mesh = plsc.VectorSubcoreMesh(core_axis_name="c", subcore_axis_name="s")

    @functools.partial(
        pl.kernel,
        mesh=mesh,
        out_type=jax.ShapeDtypeStruct((_B * _N * _K, _DS), jnp.float32),
        compiler_params=pltpu.CompilerParams(use_tc_tiling_on_sc=False),
        scratch_types=[
            pltpu.VMEM((_CPW, _CH), jnp.int32),
            pltpu.VMEM((2, _CH, _DS), jnp.float32),
            pltpu.SemaphoreType.DMA,
            pltpu.SemaphoreType.DMA,
        ],
    )
    def k(table_hbm, idx_hbm, nn_hbm, idx_v, rows_v, sem0, sem1):
        wid = lax.axis_index("s") * 2 + lax.axis_index("c")
        pltpu.sync_copy(idx_hbm.at[wid], idx_v)
        base = wid * (_CPW * _CH)
        boff = (wid // 16) * _N
        tab = table_hbm.at[pl.ds(boff, _N)]
        sems = (sem0, sem1)

        def start(j, slot):
            pltpu.async_copy(tab.at[idx_v.at[j]], rows_v.at[slot],
                             sems[slot])

        def finish(j, slot):
            pltpu.make_async_copy(tab.at[idx_v.at[j]], rows_v.at[slot],
                                  sems[slot]).wait()
            pltpu.sync_copy(rows_v.at[slot],
                            nn_hbm.at[pl.ds(base + j * _CH, _CH)])

        start(0, 0)

        def body(p, _):
            j = 2 * p
            start(j + 1, 1)
            finish(j, 0)
            start(j + 2, 0)
            finish(j + 1, 1)
            return 0

        lax.fori_loop(0, _CPW // 2 - 1, body, 0)
        start(_CPW - 1, 1)
        finish(_CPW - 2, 0)
        finish(_CPW - 1, 1)

    return k(table, idx3)


_NB = 256


def _stats_kernel(gc_ref, nn_ref, kj_ref, b1_ref, o1_ref, og_ref, oc_ref):
    step = pl.program_id(0) * pl.num_programs(1) + pl.program_id(1)

    @pl.when(step == 0)
    def _():
        o1_ref[...] = jnp.zeros_like(o1_ref)
        og_ref[...] = jnp.zeros_like(og_ref)
        oc_ref[...] = jnp.zeros_like(oc_ref)

    gcb = gc_ref[0] + b1_ref[...]
    nn = nn_ref[0]
    nnb = nn.astype(jnp.bfloat16)
    g = lax.dot_general(nnb, nnb, (((0,), (0,)), ((), ())),
                        precision=_HP,
                        preferred_element_type=jnp.float32)
    nnsum = lax.dot_general(nn, kj_ref[...], (((1,), (0,)), ((), ())),
                            precision=_HP,
                            preferred_element_type=jnp.float32)
    c = lax.dot_general(gcb, nnsum, (((0,), (0,)), ((), ())),
                        precision=_HP,
                        preferred_element_type=jnp.float32)
    a1 = jnp.sum(gcb, axis=0)
    a2 = jnp.sum(gcb * gcb, axis=0)
    t8 = jnp.sum(nnsum, axis=0)
    t8p = jnp.pad(t8, (0, _OUT - _DS))
    o1_ref[...] += jnp.stack([a1, a2, t8p])
    og_ref[...] += g
    oc_ref[...] += c


def _stats(gc, nn128, kj, b1_2):
    return pl.pallas_call(
        _stats_kernel,
        grid=(_B, _N // _NB),
        in_specs=[
            pl.BlockSpec((1, _NB, _OUT), lambda b, t: (b, t, 0)),
            pl.BlockSpec((1, _NB, _K * _DS), lambda b, t: (b, t, 0)),
            pl.BlockSpec((_K * _DS, _DS), lambda b, t: (0, 0)),
            pl.BlockSpec((1, _OUT), lambda b, t: (0, 0)),
        ],
        out_specs=[
            pl.BlockSpec((3, _OUT), lambda b, t: (0, 0)),
            pl.BlockSpec((_K * _DS, _K * _DS), lambda b, t: (0, 0)),
            pl.BlockSpec((_OUT, _DS), lambda b, t: (0, 0)),
        ],
        out_shape=[
            jax.ShapeDtypeStruct((3, _OUT), jnp.float32),
            jax.ShapeDtypeStruct((_K * _DS, _K * _DS), jnp.float32),
            jax.ShapeDtypeStruct((_OUT, _DS), jnp.float32),
        ],
    )(gc, nn128, kj, b1_2)


def _main_kernel(gc_ref, nn_ref, bd_ref, b1_ref, ae_ref, w2p_ref, b2_ref,
                 o_ref):
    gc = gc_ref[0]
    nn = nn_ref[0].astype(jnp.bfloat16)
    gn = lax.dot_general(nn, bd_ref[...], (((1,), (0,)), ((), ())),
                         precision=_HP,
                         preferred_element_type=jnp.float32)
    a = ae_ref[0][None, :]
    e = ae_ref[1][None, :]
    gcb = (gc + b1_ref[...]) * a + e
    rs = []
    for m in range(0, _K, 2):
        h0 = gcb - gn[:, m * _OUT:(m + 1) * _OUT] * a
        h1 = gcb - gn[:, (m + 1) * _OUT:(m + 2) * _OUT] * a
        rs.append(jnp.concatenate(
            [jnp.maximum(h0, 0.0), jnp.maximum(h1, 0.0)],
            axis=1).astype(jnp.bfloat16))
    r = jnp.concatenate(rs, axis=0)
    y = lax.dot_general(r, w2p_ref[...], (((1,), (0,)), ((), ())),
                        precision=_HP,
                        preferred_element_type=jnp.float32)
    ym2 = jnp.max(y.reshape(_K // 2, _NB, 2 * _OUT), axis=0)
    ym = (jnp.maximum(ym2[:, :_OUT], ym2[:, _OUT:]) + b2_ref[...])
    o_ref[0] = ym.T


def _main(gc, nn128, bd, b1_2, ae, w2p, b2_2):
    return pl.pallas_call(
        _main_kernel,
        grid=(_B, _N // _NB),
        in_specs=[
            pl.BlockSpec((1, _NB, _OUT), lambda b, t: (b, t, 0)),
            pl.BlockSpec((1, _NB, _K * _DS), lambda b, t: (b, t, 0)),
            pl.BlockSpec((_K * _DS, _K * _OUT), lambda b, t: (0, 0)),
            pl.BlockSpec((1, _OUT), lambda b, t: (0, 0)),
            pl.BlockSpec((2, _OUT), lambda b, t: (0, 0)),
            pl.BlockSpec((2 * _OUT, 2 * _OUT), lambda b, t: (0, 0)),
            pl.BlockSpec((1, _OUT), lambda b, t: (0, 0)),
        ],
        out_specs=pl.BlockSpec((1, _OUT, _NB), lambda b, t: (b, 0, t)),
        out_shape=jax.ShapeDtypeStruct((_B, _OUT, _N), jnp.float32),
    )(gc, nn128, bd, b1_2, ae, w2p, b2_2)


def kernel(feats, idxs, ds_W, ds_b, W1, b1, gamma, beta, W2, b2):
    feats3 = feats.reshape(_B, _C, _N)
    f, gc = _make_f(feats3, ds_W, ds_b.reshape(1, _DS), W1)

    idx3 = idxs.astype(jnp.int32).reshape(_NW, _CPW, _CH)
    nn = _sc_gather(f.reshape(_B * _N, _DS), idx3)
    nn128 = nn.reshape(_B, _N, _K * _DS)

    bd = (jnp.eye(_K, dtype=jnp.float32)[:, None, :, None]
          * W1.T[None, :, None, :]).reshape(_K * _DS, _K * _OUT)
    bd = bd.astype(jnp.bfloat16)
    z = jnp.zeros((_OUT, _OUT), jnp.float32)
    w2p = jnp.block([[W2.T, z], [z, W2.T]]).astype(jnp.bfloat16)

    b1_2 = b1.reshape(1, _OUT)
    kj = jnp.tile(jnp.eye(_DS, dtype=jnp.float32), (_K, 1))
    st1, stg, stc = _stats(gc, nn128, kj, b1_2)
    cnt = float(_B * _N * _K)
    a1, a2, t8 = st1[0], st1[1], st1[2, :_DS]
    m2f = sum(stg[8 * m:8 * m + 8, 8 * m:8 * m + 8] for m in range(_K))
    sum_h = _K * a1 - W1 @ t8
    term2 = jnp.sum(W1 * stc, axis=1)
    term3 = jnp.sum((W1 @ m2f) * W1, axis=1)
    sum_h2 = _K * a2 - 2.0 * term2 + term3
    mean = sum_h / cnt
    var = sum_h2 / cnt - mean * mean
    a = gamma * lax.rsqrt(var + 1e-5)
    e = beta - mean * a
    ae = jnp.stack([a, e])

    out = _main(gc, nn128, bd, b1_2, ae, w2p, b2.reshape(1, _OUT))
    return out.reshape(_B, _OUT, _N, 1)


# speedup vs baseline: 141.9733x; 1.1198x over previous
import functools

import jax
import jax.numpy as jnp
from jax import lax
from jax.experimental import pallas as pl
from jax.experimental.pallas import tpu as pltpu
from jax.experimental.pallas import tpu_sc as plsc

_B, _C, _N, _K = 2, 256, 16384, 16
_DS, _OUT = 8, 128
_HP = lax.Precision.DEFAULT


_NB1 = 2048


def _ds_kernel(feats_ref, dsw_ref, dsb_ref, w1_ref, f_ref, gc_ref):
    fb = feats_ref[0]
    w = dsw_ref[...]
    r = lax.dot_general(fb, w, (((0,), (1,)), ((), ())),
                        precision=_HP,
                        preferred_element_type=jnp.float32)
    r = r + dsb_ref[...]
    f_ref[0] = r
    gc_ref[0] = lax.dot_general(r, w1_ref[...], (((1,), (1,)), ((), ())),
                                precision=_HP,
                                preferred_element_type=jnp.float32)


def _make_f(feats, ds_W, ds_b2, W1):
    return pl.pallas_call(
        _ds_kernel,
        grid=(_B, _N // _NB1),
        in_specs=[
            pl.BlockSpec((1, _C, _NB1), lambda b, t: (b, 0, t)),
            pl.BlockSpec((_DS, _C), lambda b, t: (0, 0)),
            pl.BlockSpec((1, _DS), lambda b, t: (0, 0)),
            pl.BlockSpec((_OUT, _DS), lambda b, t: (0, 0)),
        ],
        out_specs=[
            pl.BlockSpec((1, _NB1, _DS), lambda b, t: (b, t, 0)),
            pl.BlockSpec((1, _NB1, _OUT), lambda b, t: (b, t, 0)),
        ],
        out_shape=[
            jax.ShapeDtypeStruct((_B, _N, _DS), jnp.float32),
            jax.ShapeDtypeStruct((_B, _N, _OUT), jnp.float32),
        ],
    )(feats, ds_W, ds_b2, W1)


_NW = 32
_CH = 128
_CPW = (_B * _N * _K) // (_NW * _CH)
_G = 4
_NG = _CPW // _G


def _sc_gather(table, idx_flat):
    mesh = plsc.VectorSubcoreMesh(core_axis_name="c", subcore_axis_name="s")

    @functools.partial(
        pl.kernel,
        mesh=mesh,
        out_type=jax.ShapeDtypeStruct((_B * _N * _K, _DS), jnp.float32),
        compiler_params=pltpu.CompilerParams(use_tc_tiling_on_sc=False),
        scratch_types=[
            pltpu.VMEM((_CPW * _CH,), jnp.int32),
            pltpu.VMEM((2, _G * _CH, _DS), jnp.float32),
            pltpu.SemaphoreType.DMA,
            pltpu.SemaphoreType.DMA,
        ],
    )
    def k(table_hbm, idx_hbm, nn_hbm, idx_v, rows_v, sem0, sem1):
        wid = lax.axis_index("s") * 2 + lax.axis_index("c")
        base = wid * (_CPW * _CH)
        pltpu.sync_copy(idx_hbm.at[pl.ds(base, _CPW * _CH)], idx_v)
        boff = (wid // 16) * _N
        tab = table_hbm.at[pl.ds(boff, _N)]
        sems = (sem0, sem1)

        def startg(g, slot):
            for i in range(_G):
                pltpu.async_copy(
                    tab.at[idx_v.at[pl.ds((g * _G + i) * _CH, _CH)]],
                    rows_v.at[slot, pl.ds(i * _CH, _CH)], sems[slot])

        def draing(g, slot):
            for i in range(_G):
                pltpu.make_async_copy(
                    tab.at[idx_v.at[pl.ds((g * _G + i) * _CH, _CH)]],
                    rows_v.at[slot, pl.ds(i * _CH, _CH)], sems[slot]).wait()

        def wb(g, slot):
            pltpu.sync_copy(rows_v.at[slot],
                            nn_hbm.at[pl.ds(base + g * _G * _CH, _G * _CH)])

        startg(0, 0)

        def body(p, _):
            g = 2 * p
            startg(g + 1, 1)
            draing(g, 0)
            wb(g, 0)
            startg(g + 2, 0)
            draing(g + 1, 1)
            wb(g + 1, 1)
            return 0

        lax.fori_loop(0, _NG // 2 - 1, body, 0)
        g_last = _NG - 2
        startg(g_last + 1, 1)
        draing(g_last, 0)
        wb(g_last, 0)
        draing(g_last + 1, 1)
        wb(g_last + 1, 1)

    return k(table, idx_flat)


_NB = 256


def _stats_kernel(gc_ref, nn_ref, kj_ref, b1_ref, ones_ref, o1_ref, og_ref,
                  oc_ref):
    step = pl.program_id(0) * pl.num_programs(1) + pl.program_id(1)

    @pl.when(step == 0)
    def _():
        o1_ref[...] = jnp.zeros_like(o1_ref)
        og_ref[...] = jnp.zeros_like(og_ref)
        oc_ref[...] = jnp.zeros_like(oc_ref)

    gcb = gc_ref[0] + b1_ref[...]
    nn = nn_ref[0]
    nnb = nn.astype(jnp.bfloat16)
    g = lax.dot_general(nnb, nnb, (((0,), (0,)), ((), ())),
                        precision=_HP,
                        preferred_element_type=jnp.float32)
    nnsum = lax.dot_general(nn, kj_ref[...], (((1,), (0,)), ((), ())),
                            precision=_HP,
                            preferred_element_type=jnp.float32)
    c = lax.dot_general(gcb, nnsum, (((0,), (0,)), ((), ())),
                        precision=_HP,
                        preferred_element_type=jnp.float32)
    ones = ones_ref[...]
    a1 = lax.dot_general(ones, gcb, (((1,), (0,)), ((), ())),
                         precision=_HP,
                         preferred_element_type=jnp.float32)
    a2 = lax.dot_general(ones, gcb * gcb, (((1,), (0,)), ((), ())),
                         precision=_HP,
                         preferred_element_type=jnp.float32)
    t8 = lax.dot_general(ones, nnsum, (((1,), (0,)), ((), ())),
                         precision=_HP,
                         preferred_element_type=jnp.float32)
    t8p = jnp.pad(t8, ((0, 0), (0, _OUT - _DS)))
    o1_ref[...] += jnp.concatenate([a1, a2, t8p], axis=0)
    og_ref[...] += g
    oc_ref[...] += c


def _stats(gc, nn128, kj, b1_2, ones):
    return pl.pallas_call(
        _stats_kernel,
        grid=(_B, _N // _NB),
        in_specs=[
            pl.BlockSpec((1, _NB, _OUT), lambda b, t: (b, t, 0)),
            pl.BlockSpec((1, _NB, _K * _DS), lambda b, t: (b, t, 0)),
            pl.BlockSpec((_K * _DS, _DS), lambda b, t: (0, 0)),
            pl.BlockSpec((1, _OUT), lambda b, t: (0, 0)),
            pl.BlockSpec((1, _NB), lambda b, t: (0, 0)),
        ],
        out_specs=[
            pl.BlockSpec((3, _OUT), lambda b, t: (0, 0)),
            pl.BlockSpec((_K * _DS, _K * _DS), lambda b, t: (0, 0)),
            pl.BlockSpec((_OUT, _DS), lambda b, t: (0, 0)),
        ],
        out_shape=[
            jax.ShapeDtypeStruct((3, _OUT), jnp.float32),
            jax.ShapeDtypeStruct((_K * _DS, _K * _DS), jnp.float32),
            jax.ShapeDtypeStruct((_OUT, _DS), jnp.float32),
        ],
    )(gc, nn128, kj, b1_2, ones)


def _main_kernel(gc_ref, nn_ref, bd_ref, b1_ref, ae_ref, w2p_ref, b2_ref,
                 o_ref):
    gc = gc_ref[0]
    nn = nn_ref[0].astype(jnp.bfloat16)
    gn = lax.dot_general(nn, bd_ref[...], (((1,), (0,)), ((), ())),
                         precision=_HP,
                         preferred_element_type=jnp.float32
                         ).astype(jnp.bfloat16)
    a = ae_ref[0][None, :]
    e = ae_ref[1][None, :]
    ab = a.astype(jnp.bfloat16)
    gcb = ((gc + b1_ref[...]) * a + e).astype(jnp.bfloat16)
    zero = jnp.bfloat16(0.0)
    rs = []
    for m in range(0, _K, 2):
        h0 = gcb - gn[:, m * _OUT:(m + 1) * _OUT] * ab
        h1 = gcb - gn[:, (m + 1) * _OUT:(m + 2) * _OUT] * ab
        rs.append(jnp.concatenate(
            [jnp.maximum(h0, zero), jnp.maximum(h1, zero)], axis=1))
    r = jnp.concatenate(rs, axis=0)
    y = lax.dot_general(r, w2p_ref[...], (((1,), (0,)), ((), ())),
                        precision=_HP,
                        preferred_element_type=jnp.float32)
    ym2 = jnp.max(y.reshape(_K // 2, _NB, 2 * _OUT), axis=0)
    ym = (jnp.maximum(ym2[:, :_OUT], ym2[:, _OUT:]) + b2_ref[...])
    o_ref[0] = ym.T


def _main(gc, nn128, bd, b1_2, ae, w2p, b2_2):
    return pl.pallas_call(
        _main_kernel,
        grid=(_B, _N // _NB),
        in_specs=[
            pl.BlockSpec((1, _NB, _OUT), lambda b, t: (b, t, 0)),
            pl.BlockSpec((1, _NB, _K * _DS), lambda b, t: (b, t, 0)),
            pl.BlockSpec((_K * _DS, _K * _OUT), lambda b, t: (0, 0)),
            pl.BlockSpec((1, _OUT), lambda b, t: (0, 0)),
            pl.BlockSpec((2, _OUT), lambda b, t: (0, 0)),
            pl.BlockSpec((2 * _OUT, 2 * _OUT), lambda b, t: (0, 0)),
            pl.BlockSpec((1, _OUT), lambda b, t: (0, 0)),
        ],
        out_specs=pl.BlockSpec((1, _OUT, _NB), lambda b, t: (b, 0, t)),
        out_shape=jax.ShapeDtypeStruct((_B, _OUT, _N), jnp.float32),
    )(gc, nn128, bd, b1_2, ae, w2p, b2_2)


def kernel(feats, idxs, ds_W, ds_b, W1, b1, gamma, beta, W2, b2):
    feats3 = feats.reshape(_B, _C, _N)
    f, gc = _make_f(feats3, ds_W, ds_b.reshape(1, _DS), W1)

    idx_flat = idxs.astype(jnp.int32).reshape(-1)
    nn = _sc_gather(f.reshape(_B * _N, _DS), idx_flat)
    nn128 = nn.reshape(_B, _N, _K * _DS)

    bd = (jnp.eye(_K, dtype=jnp.float32)[:, None, :, None]
          * W1.T[None, :, None, :]).reshape(_K * _DS, _K * _OUT)
    bd = bd.astype(jnp.bfloat16)
    z = jnp.zeros((_OUT, _OUT), jnp.float32)
    w2p = jnp.block([[W2.T, z], [z, W2.T]]).astype(jnp.bfloat16)

    b1_2 = b1.reshape(1, _OUT)
    kj = jnp.tile(jnp.eye(_DS, dtype=jnp.float32), (_K, 1))
    ones = jnp.ones((1, _NB), jnp.float32)
    st1, stg, stc = _stats(gc, nn128, kj, b1_2, ones)
    cnt = float(_B * _N * _K)
    a1, a2, t8 = st1[0], st1[1], st1[2, :_DS]
    m2f = sum(stg[8 * m:8 * m + 8, 8 * m:8 * m + 8] for m in range(_K))
    sum_h = _K * a1 - W1 @ t8
    term2 = jnp.sum(W1 * stc, axis=1)
    term3 = jnp.sum((W1 @ m2f) * W1, axis=1)
    sum_h2 = _K * a2 - 2.0 * term2 + term3
    mean = sum_h / cnt
    var = sum_h2 / cnt - mean * mean
    a = gamma * lax.rsqrt(var + 1e-5)
    e = beta - mean * a
    ae = jnp.stack([a, e])

    out = _main(gc, nn128, bd, b1_2, ae, w2p, b2.reshape(1, _OUT))
    return out.reshape(_B, _OUT, _N, 1)


# speedup vs baseline: 170.9374x; 1.2040x over previous
import functools

import jax
import jax.numpy as jnp
from jax import lax
from jax.experimental import pallas as pl
from jax.experimental.pallas import tpu as pltpu
from jax.experimental.pallas import tpu_sc as plsc

_B, _C, _N, _K = 2, 256, 16384, 16
_DS, _OUT = 8, 128
_HP = lax.Precision.DEFAULT


_NB1 = 2048


def _ds_kernel(feats_ref, dsw_ref, dsb_ref, w1_ref, f_ref, gc_ref):
    fb = feats_ref[0]
    w = dsw_ref[...]
    r = lax.dot_general(fb, w, (((0,), (1,)), ((), ())),
                        precision=_HP,
                        preferred_element_type=jnp.float32)
    r = r + dsb_ref[...]
    f_ref[0] = r
    gc_ref[0] = lax.dot_general(r, w1_ref[...], (((1,), (1,)), ((), ())),
                                precision=_HP,
                                preferred_element_type=jnp.float32)


def _make_f(feats, ds_W, ds_b2, W1):
    return pl.pallas_call(
        _ds_kernel,
        grid=(_B, _N // _NB1),
        in_specs=[
            pl.BlockSpec((1, _C, _NB1), lambda b, t: (b, 0, t)),
            pl.BlockSpec((_DS, _C), lambda b, t: (0, 0)),
            pl.BlockSpec((1, _DS), lambda b, t: (0, 0)),
            pl.BlockSpec((_OUT, _DS), lambda b, t: (0, 0)),
        ],
        out_specs=[
            pl.BlockSpec((1, _NB1, _DS), lambda b, t: (b, t, 0)),
            pl.BlockSpec((1, _NB1, _OUT), lambda b, t: (b, t, 0)),
        ],
        out_shape=[
            jax.ShapeDtypeStruct((_B, _N, _DS), jnp.float32),
            jax.ShapeDtypeStruct((_B, _N, _OUT), jnp.float32),
        ],
    )(feats, ds_W, ds_b2, W1)


_NW = 32
_CH = 128
_CPW = (_B * _N * _K) // (_NW * _CH)
_G = 4
_NG = _CPW // _G


def _sc_gather(table, idx_flat):
    mesh = plsc.VectorSubcoreMesh(core_axis_name="c", subcore_axis_name="s")

    @functools.partial(
        pl.kernel,
        mesh=mesh,
        out_type=jax.ShapeDtypeStruct((_B * _N * _K, _DS), jnp.float32),
        compiler_params=pltpu.CompilerParams(use_tc_tiling_on_sc=False),
        scratch_types=[
            pltpu.VMEM((_CPW * _CH,), jnp.int32),
            pltpu.VMEM((2, _G * _CH, _DS), jnp.float32),
            pltpu.SemaphoreType.DMA,
            pltpu.SemaphoreType.DMA,
        ],
    )
    def k(table_hbm, idx_hbm, nn_hbm, idx_v, rows_v, sem0, sem1):
        wid = lax.axis_index("s") * 2 + lax.axis_index("c")
        base = wid * (_CPW * _CH)
        pltpu.sync_copy(idx_hbm.at[pl.ds(base, _CPW * _CH)], idx_v)
        boff = (wid // 16) * _N
        tab = table_hbm.at[pl.ds(boff, _N)]
        sems = (sem0, sem1)

        def startg(g, slot):
            for i in range(_G):
                pltpu.async_copy(
                    tab.at[idx_v.at[pl.ds((g * _G + i) * _CH, _CH)]],
                    rows_v.at[slot, pl.ds(i * _CH, _CH)], sems[slot])

        def draing(g, slot):
            for i in range(_G):
                pltpu.make_async_copy(
                    tab.at[idx_v.at[pl.ds((g * _G + i) * _CH, _CH)]],
                    rows_v.at[slot, pl.ds(i * _CH, _CH)], sems[slot]).wait()

        def wb(g, slot):
            pltpu.sync_copy(rows_v.at[slot],
                            nn_hbm.at[pl.ds(base + g * _G * _CH, _G * _CH)])

        startg(0, 0)

        def body(p, _):
            g = 2 * p
            startg(g + 1, 1)
            draing(g, 0)
            wb(g, 0)
            startg(g + 2, 0)
            draing(g + 1, 1)
            wb(g + 1, 1)
            return 0

        lax.fori_loop(0, _NG // 2 - 1, body, 0)
        g_last = _NG - 2
        startg(g_last + 1, 1)
        draing(g_last, 0)
        wb(g_last, 0)
        draing(g_last + 1, 1)
        wb(g_last + 1, 1)

    return k(table, idx_flat)


_NB = 512


def _stats_kernel(gc_ref, nn_ref, kj_ref, b1_ref, ones_ref, o1_ref, og_ref,
                  oc_ref):
    step = pl.program_id(0) * pl.num_programs(1) + pl.program_id(1)

    @pl.when(step == 0)
    def _():
        o1_ref[...] = jnp.zeros_like(o1_ref)
        og_ref[...] = jnp.zeros_like(og_ref)
        oc_ref[...] = jnp.zeros_like(oc_ref)

    gcb = gc_ref[0] + b1_ref[...]
    nn = nn_ref[0]
    nnb = nn.astype(jnp.bfloat16)
    g = lax.dot_general(nnb, nnb, (((0,), (0,)), ((), ())),
                        precision=_HP,
                        preferred_element_type=jnp.float32)
    nnsum = lax.dot_general(nn, kj_ref[...], (((1,), (0,)), ((), ())),
                            precision=_HP,
                            preferred_element_type=jnp.float32)
    c = lax.dot_general(gcb, nnsum, (((0,), (0,)), ((), ())),
                        precision=_HP,
                        preferred_element_type=jnp.float32)
    ones = ones_ref[...]
    a1 = lax.dot_general(ones, gcb, (((1,), (0,)), ((), ())),
                         precision=_HP,
                         preferred_element_type=jnp.float32)
    a2 = lax.dot_general(ones, gcb * gcb, (((1,), (0,)), ((), ())),
                         precision=_HP,
                         preferred_element_type=jnp.float32)
    t8 = lax.dot_general(ones, nnsum, (((1,), (0,)), ((), ())),
                         precision=_HP,
                         preferred_element_type=jnp.float32)
    t8p = jnp.pad(t8, ((0, 0), (0, _OUT - _DS)))
    o1_ref[...] += jnp.concatenate([a1, a2, t8p], axis=0)
    og_ref[...] += g
    oc_ref[...] += c


def _stats(gc, nn128, kj, b1_2, ones):
    return pl.pallas_call(
        _stats_kernel,
        grid=(_B, _N // _NB),
        in_specs=[
            pl.BlockSpec((1, _NB, _OUT), lambda b, t: (b, t, 0)),
            pl.BlockSpec((1, _NB, _K * _DS), lambda b, t: (b, t, 0)),
            pl.BlockSpec((_K * _DS, _DS), lambda b, t: (0, 0)),
            pl.BlockSpec((1, _OUT), lambda b, t: (0, 0)),
            pl.BlockSpec((1, _NB), lambda b, t: (0, 0)),
        ],
        out_specs=[
            pl.BlockSpec((3, _OUT), lambda b, t: (0, 0)),
            pl.BlockSpec((_K * _DS, _K * _DS), lambda b, t: (0, 0)),
            pl.BlockSpec((_OUT, _DS), lambda b, t: (0, 0)),
        ],
        out_shape=[
            jax.ShapeDtypeStruct((3, _OUT), jnp.float32),
            jax.ShapeDtypeStruct((_K * _DS, _K * _DS), jnp.float32),
            jax.ShapeDtypeStruct((_OUT, _DS), jnp.float32),
        ],
    )(gc, nn128, kj, b1_2, ones)


def _main_kernel(gc_ref, nn_ref, bd_ref, b1_ref, ae_ref, w2p_ref, b2_ref,
                 o_ref):
    gc = gc_ref[0]
    nn = nn_ref[0].astype(jnp.bfloat16)
    gn = lax.dot_general(nn, bd_ref[...], (((1,), (0,)), ((), ())),
                         precision=_HP,
                         preferred_element_type=jnp.float32
                         ).astype(jnp.bfloat16)
    a = ae_ref[0][None, :]
    e = ae_ref[1][None, :]
    ab = a.astype(jnp.bfloat16)
    gcb = ((gc + b1_ref[...]) * a + e).astype(jnp.bfloat16)
    zero = jnp.bfloat16(0.0)
    rs = []
    for m in range(0, _K, 2):
        h0 = gcb - gn[:, m * _OUT:(m + 1) * _OUT] * ab
        h1 = gcb - gn[:, (m + 1) * _OUT:(m + 2) * _OUT] * ab
        rs.append(jnp.concatenate(
            [jnp.maximum(h0, zero), jnp.maximum(h1, zero)], axis=1))
    r = jnp.concatenate(rs, axis=0)
    y = lax.dot_general(r, w2p_ref[...], (((1,), (0,)), ((), ())),
                        precision=_HP,
                        preferred_element_type=jnp.float32)
    ym2 = jnp.max(y.reshape(_K // 2, _NB, 2 * _OUT), axis=0)
    ym = (jnp.maximum(ym2[:, :_OUT], ym2[:, _OUT:]) + b2_ref[...])
    o_ref[0] = ym.T


def _main(gc, nn128, bd, b1_2, ae, w2p, b2_2):
    return pl.pallas_call(
        _main_kernel,
        grid=(_B, _N // _NB),
        in_specs=[
            pl.BlockSpec((1, _NB, _OUT), lambda b, t: (b, t, 0)),
            pl.BlockSpec((1, _NB, _K * _DS), lambda b, t: (b, t, 0)),
            pl.BlockSpec((_K * _DS, _K * _OUT), lambda b, t: (0, 0)),
            pl.BlockSpec((1, _OUT), lambda b, t: (0, 0)),
            pl.BlockSpec((2, _OUT), lambda b, t: (0, 0)),
            pl.BlockSpec((2 * _OUT, 2 * _OUT), lambda b, t: (0, 0)),
            pl.BlockSpec((1, _OUT), lambda b, t: (0, 0)),
        ],
        out_specs=pl.BlockSpec((1, _OUT, _NB), lambda b, t: (b, 0, t)),
        out_shape=jax.ShapeDtypeStruct((_B, _OUT, _N), jnp.float32),
    )(gc, nn128, bd, b1_2, ae, w2p, b2_2)


def kernel(feats, idxs, ds_W, ds_b, W1, b1, gamma, beta, W2, b2):
    feats3 = feats.reshape(_B, _C, _N)
    f, gc = _make_f(feats3, ds_W, ds_b.reshape(1, _DS), W1)

    idx_flat = idxs.astype(jnp.int32).reshape(-1)
    nn = _sc_gather(f.reshape(_B * _N, _DS), idx_flat)
    nn128 = nn.reshape(_B, _N, _K * _DS)

    bd = (jnp.eye(_K, dtype=jnp.float32)[:, None, :, None]
          * W1.T[None, :, None, :]).reshape(_K * _DS, _K * _OUT)
    bd = bd.astype(jnp.bfloat16)
    z = jnp.zeros((_OUT, _OUT), jnp.float32)
    w2p = jnp.block([[W2.T, z], [z, W2.T]]).astype(jnp.bfloat16)

    b1_2 = b1.reshape(1, _OUT)
    kj = jnp.tile(jnp.eye(_DS, dtype=jnp.float32), (_K, 1))
    ones = jnp.ones((1, _NB), jnp.float32)
    st1, stg, stc = _stats(gc, nn128, kj, b1_2, ones)
    cnt = float(_B * _N * _K)
    a1, a2, t8 = st1[0], st1[1], st1[2, :_DS]
    m2f = sum(stg[8 * m:8 * m + 8, 8 * m:8 * m + 8] for m in range(_K))
    sum_h = _K * a1 - W1 @ t8
    term2 = jnp.sum(W1 * stc, axis=1)
    term3 = jnp.sum((W1 @ m2f) * W1, axis=1)
    sum_h2 = _K * a2 - 2.0 * term2 + term3
    mean = sum_h / cnt
    var = sum_h2 / cnt - mean * mean
    a = gamma * lax.rsqrt(var + 1e-5)
    e = beta - mean * a
    ae = jnp.stack([a, e])

    out = _main(gc, nn128, bd, b1_2, ae, w2p, b2.reshape(1, _OUT))
    return out.reshape(_B, _OUT, _N, 1)


# speedup vs baseline: 190.5495x; 1.1147x over previous
import functools

import jax
import jax.numpy as jnp
from jax import lax
from jax.experimental import pallas as pl
from jax.experimental.pallas import tpu as pltpu
from jax.experimental.pallas import tpu_sc as plsc

_B, _C, _N, _K = 2, 256, 16384, 16
_DS, _OUT = 8, 128
_HP = lax.Precision.DEFAULT


_NB1 = 2048


def _ds_kernel(feats_ref, dsw_ref, dsb_ref, w1_ref, f_ref, gc_ref):
    fb = feats_ref[0]
    w = dsw_ref[...]
    r = lax.dot_general(fb, w, (((0,), (1,)), ((), ())),
                        precision=_HP,
                        preferred_element_type=jnp.float32)
    r = r + dsb_ref[...]
    f_ref[0] = r
    gc_ref[0] = lax.dot_general(r, w1_ref[...], (((1,), (1,)), ((), ())),
                                precision=_HP,
                                preferred_element_type=jnp.float32)


def _make_f(feats, ds_W, ds_b2, W1):
    return pl.pallas_call(
        _ds_kernel,
        grid=(_B, _N // _NB1),
        in_specs=[
            pl.BlockSpec((1, _C, _NB1), lambda b, t: (b, 0, t)),
            pl.BlockSpec((_DS, _C), lambda b, t: (0, 0)),
            pl.BlockSpec((1, _DS), lambda b, t: (0, 0)),
            pl.BlockSpec((_OUT, _DS), lambda b, t: (0, 0)),
        ],
        out_specs=[
            pl.BlockSpec((1, _NB1, _DS), lambda b, t: (b, t, 0)),
            pl.BlockSpec((1, _NB1, _OUT), lambda b, t: (b, t, 0)),
        ],
        out_shape=[
            jax.ShapeDtypeStruct((_B, _N, _DS), jnp.float32),
            jax.ShapeDtypeStruct((_B, _N, _OUT), jnp.float32),
        ],
    )(feats, ds_W, ds_b2, W1)


_NW = 32
_CH = 128
_CPW = (_B * _N * _K) // (_NW * _CH)
_G = 4
_NG = _CPW // _G


def _sc_gather(table, idx_flat):
    mesh = plsc.VectorSubcoreMesh(core_axis_name="c", subcore_axis_name="s")

    @functools.partial(
        pl.kernel,
        mesh=mesh,
        out_type=jax.ShapeDtypeStruct((_B * _N * _K, _DS), jnp.float32),
        compiler_params=pltpu.CompilerParams(use_tc_tiling_on_sc=False),
        scratch_types=[
            pltpu.VMEM((_CPW * _CH,), jnp.int32),
            pltpu.VMEM((2, _G * _CH, _DS), jnp.float32),
            pltpu.SemaphoreType.DMA,
            pltpu.SemaphoreType.DMA,
        ],
    )
    def k(table_hbm, idx_hbm, nn_hbm, idx_v, rows_v, sem0, sem1):
        wid = lax.axis_index("s") * 2 + lax.axis_index("c")
        base = wid * (_CPW * _CH)
        pltpu.sync_copy(idx_hbm.at[pl.ds(base, _CPW * _CH)], idx_v)
        boff = (wid // 16) * _N
        tab = table_hbm.at[pl.ds(boff, _N)]
        sems = (sem0, sem1)

        def startg(g, slot):
            for i in range(_G):
                pltpu.async_copy(
                    tab.at[idx_v.at[pl.ds((g * _G + i) * _CH, _CH)]],
                    rows_v.at[slot, pl.ds(i * _CH, _CH)], sems[slot])

        def draing(g, slot):
            for i in range(_G):
                pltpu.make_async_copy(
                    tab.at[idx_v.at[pl.ds((g * _G + i) * _CH, _CH)]],
                    rows_v.at[slot, pl.ds(i * _CH, _CH)], sems[slot]).wait()

        def wb(g, slot):
            pltpu.sync_copy(rows_v.at[slot],
                            nn_hbm.at[pl.ds(base + g * _G * _CH, _G * _CH)])

        startg(0, 0)

        def body(p, _):
            g = 2 * p
            startg(g + 1, 1)
            draing(g, 0)
            wb(g, 0)
            startg(g + 2, 0)
            draing(g + 1, 1)
            wb(g + 1, 1)
            return 0

        lax.fori_loop(0, _NG // 2 - 1, body, 0)
        g_last = _NG - 2
        startg(g_last + 1, 1)
        draing(g_last, 0)
        wb(g_last, 0)
        draing(g_last + 1, 1)
        wb(g_last + 1, 1)

    return k(table, idx_flat)


_NB = 1024


def _stats_kernel(gc_ref, nn_ref, kj_ref, b1_ref, ones_ref, o1_ref, og_ref,
                  oc_ref):
    step = pl.program_id(0) * pl.num_programs(1) + pl.program_id(1)

    @pl.when(step == 0)
    def _():
        o1_ref[...] = jnp.zeros_like(o1_ref)
        og_ref[...] = jnp.zeros_like(og_ref)
        oc_ref[...] = jnp.zeros_like(oc_ref)

    gcb = gc_ref[0] + b1_ref[...]
    nn = nn_ref[0]
    nnb = nn.astype(jnp.bfloat16)
    g = lax.dot_general(nnb, nnb, (((0,), (0,)), ((), ())),
                        precision=_HP,
                        preferred_element_type=jnp.float32)
    nnsum = lax.dot_general(nn, kj_ref[...], (((1,), (0,)), ((), ())),
                            precision=_HP,
                            preferred_element_type=jnp.float32)
    c = lax.dot_general(gcb, nnsum, (((0,), (0,)), ((), ())),
                        precision=_HP,
                        preferred_element_type=jnp.float32)
    ones = ones_ref[...]
    a1 = lax.dot_general(ones, gcb, (((1,), (0,)), ((), ())),
                         precision=_HP,
                         preferred_element_type=jnp.float32)
    a2 = lax.dot_general(ones, gcb * gcb, (((1,), (0,)), ((), ())),
                         precision=_HP,
                         preferred_element_type=jnp.float32)
    t8 = lax.dot_general(ones, nnsum, (((1,), (0,)), ((), ())),
                         precision=_HP,
                         preferred_element_type=jnp.float32)
    t8p = jnp.pad(t8, ((0, 0), (0, _OUT - _DS)))
    o1_ref[...] += jnp.concatenate([a1, a2, t8p], axis=0)
    og_ref[...] += g
    oc_ref[...] += c


def _stats(gc, nn128, kj, b1_2, ones):
    return pl.pallas_call(
        _stats_kernel,
        grid=(_B, _N // _NB),
        in_specs=[
            pl.BlockSpec((1, _NB, _OUT), lambda b, t: (b, t, 0)),
            pl.BlockSpec((1, _NB, _K * _DS), lambda b, t: (b, t, 0)),
            pl.BlockSpec((_K * _DS, _DS), lambda b, t: (0, 0)),
            pl.BlockSpec((1, _OUT), lambda b, t: (0, 0)),
            pl.BlockSpec((1, _NB), lambda b, t: (0, 0)),
        ],
        out_specs=[
            pl.BlockSpec((3, _OUT), lambda b, t: (0, 0)),
            pl.BlockSpec((_K * _DS, _K * _DS), lambda b, t: (0, 0)),
            pl.BlockSpec((_OUT, _DS), lambda b, t: (0, 0)),
        ],
        out_shape=[
            jax.ShapeDtypeStruct((3, _OUT), jnp.float32),
            jax.ShapeDtypeStruct((_K * _DS, _K * _DS), jnp.float32),
            jax.ShapeDtypeStruct((_OUT, _DS), jnp.float32),
        ],
    )(gc, nn128, kj, b1_2, ones)


def _main_kernel(gc_ref, nn_ref, bd_ref, b1_ref, ae_ref, w2p_ref, b2_ref,
                 o_ref):
    gc = gc_ref[0]
    nn = nn_ref[0].astype(jnp.bfloat16)
    gn = lax.dot_general(nn, bd_ref[...], (((1,), (0,)), ((), ())),
                         precision=_HP,
                         preferred_element_type=jnp.float32
                         ).astype(jnp.bfloat16)
    a = ae_ref[0][None, :]
    e = ae_ref[1][None, :]
    ab = a.astype(jnp.bfloat16)
    gcb = ((gc + b1_ref[...]) * a + e).astype(jnp.bfloat16)
    zero = jnp.bfloat16(0.0)
    rs = []
    for m in range(0, _K, 2):
        h0 = gcb - gn[:, m * _OUT:(m + 1) * _OUT] * ab
        h1 = gcb - gn[:, (m + 1) * _OUT:(m + 2) * _OUT] * ab
        rs.append(jnp.concatenate(
            [jnp.maximum(h0, zero), jnp.maximum(h1, zero)], axis=1))
    r = jnp.concatenate(rs, axis=0)
    y = lax.dot_general(r, w2p_ref[...], (((1,), (0,)), ((), ())),
                        precision=_HP,
                        preferred_element_type=jnp.float32)
    ym2 = jnp.max(y.reshape(_K // 2, _NB, 2 * _OUT), axis=0)
    ym = (jnp.maximum(ym2[:, :_OUT], ym2[:, _OUT:]) + b2_ref[...])
    o_ref[0] = ym.T


def _main(gc, nn128, bd, b1_2, ae, w2p, b2_2):
    return pl.pallas_call(
        _main_kernel,
        grid=(_B, _N // _NB),
        in_specs=[
            pl.BlockSpec((1, _NB, _OUT), lambda b, t: (b, t, 0)),
            pl.BlockSpec((1, _NB, _K * _DS), lambda b, t: (b, t, 0)),
            pl.BlockSpec((_K * _DS, _K * _OUT), lambda b, t: (0, 0)),
            pl.BlockSpec((1, _OUT), lambda b, t: (0, 0)),
            pl.BlockSpec((2, _OUT), lambda b, t: (0, 0)),
            pl.BlockSpec((2 * _OUT, 2 * _OUT), lambda b, t: (0, 0)),
            pl.BlockSpec((1, _OUT), lambda b, t: (0, 0)),
        ],
        out_specs=pl.BlockSpec((1, _OUT, _NB), lambda b, t: (b, 0, t)),
        out_shape=jax.ShapeDtypeStruct((_B, _OUT, _N), jnp.float32),
    )(gc, nn128, bd, b1_2, ae, w2p, b2_2)


def kernel(feats, idxs, ds_W, ds_b, W1, b1, gamma, beta, W2, b2):
    feats3 = feats.reshape(_B, _C, _N)
    f, gc = _make_f(feats3, ds_W, ds_b.reshape(1, _DS), W1)

    idx_flat = idxs.astype(jnp.int32).reshape(-1)
    nn = _sc_gather(f.reshape(_B * _N, _DS), idx_flat)
    nn128 = nn.reshape(_B, _N, _K * _DS)

    bd = (jnp.eye(_K, dtype=jnp.float32)[:, None, :, None]
          * W1.T[None, :, None, :]).reshape(_K * _DS, _K * _OUT)
    bd = bd.astype(jnp.bfloat16)
    z = jnp.zeros((_OUT, _OUT), jnp.float32)
    w2p = jnp.block([[W2.T, z], [z, W2.T]]).astype(jnp.bfloat16)

    b1_2 = b1.reshape(1, _OUT)
    kj = jnp.tile(jnp.eye(_DS, dtype=jnp.float32), (_K, 1))
    ones = jnp.ones((1, _NB), jnp.float32)
    st1, stg, stc = _stats(gc, nn128, kj, b1_2, ones)
    cnt = float(_B * _N * _K)
    a1, a2, t8 = st1[0], st1[1], st1[2, :_DS]
    m2f = sum(stg[8 * m:8 * m + 8, 8 * m:8 * m + 8] for m in range(_K))
    sum_h = _K * a1 - W1 @ t8
    term2 = jnp.sum(W1 * stc, axis=1)
    term3 = jnp.sum((W1 @ m2f) * W1, axis=1)
    sum_h2 = _K * a2 - 2.0 * term2 + term3
    mean = sum_h / cnt
    var = sum_h2 / cnt - mean * mean
    a = gamma * lax.rsqrt(var + 1e-5)
    e = beta - mean * a
    ae = jnp.stack([a, e])

    out = _main(gc, nn128, bd, b1_2, ae, w2p, b2.reshape(1, _OUT))
    return out.reshape(_B, _OUT, _N, 1)
